# baseline (speedup 1.0000x reference)
import functools
import math

import jax
import jax.numpy as jnp
from jax.experimental import pallas as pl

D = 1024
H = 16
DH = D // H
NROT = DH // 2
E = 64
F = 128
K = 2
S = 2048
ROPE_BASE = 10000.0
SB = 256
NSB = S // SB


def _ln(x, g, b):
    m = jnp.mean(x, axis=-1, keepdims=True)
    v = jnp.mean((x - m) ** 2, axis=-1, keepdims=True)
    return (x - m) * jax.lax.rsqrt(v + 1e-5) * g + b


def _qkv_body(x_ref, g_ref, b_ref, wq_ref, wk_ref, wv_ref, c_ref, s_ref,
              q_ref, k_ref, v_ref):
    xb = x_ref[...]
    nx = _ln(xb, g_ref[...], b_ref[...]).astype(jnp.bfloat16)
    q = jnp.dot(nx, wq_ref[...], preferred_element_type=jnp.float32)
    k = jnp.dot(nx, wk_ref[...], preferred_element_type=jnp.float32)
    v = jnp.dot(nx, wv_ref[...], preferred_element_type=jnp.float32)
    cb = c_ref[...]
    sb = s_ref[...]
    cfull = jnp.concatenate([cb] * H, axis=1)
    sfull = jnp.concatenate([sb] * H, axis=1)
    lane = jax.lax.broadcasted_iota(jnp.int32, (SB, D), 1)
    even = (lane % 2) == 0
    qs = jnp.where(even, jnp.roll(q, -1, axis=1), jnp.roll(q, 1, axis=1))
    ks = jnp.where(even, jnp.roll(k, -1, axis=1), jnp.roll(k, 1, axis=1))
    q_ref[...] = q * cfull + qs * sfull
    k_ref[...] = k * cfull + ks * sfull
    v_ref[...] = v


def _attn_body(q_ref, k_ref, v_ref, o_ref):
    qb = q_ref[...]
    kb = k_ref[...]
    vb = v_ref[...]
    outs = []
    for h in range(H):
        qh = qb[:, h * DH:(h + 1) * DH].astype(jnp.bfloat16)
        kh = kb[:, h * DH:(h + 1) * DH].astype(jnp.bfloat16)
        vh = vb[:, h * DH:(h + 1) * DH].astype(jnp.bfloat16)
        s = jax.lax.dot_general(qh, kh, (((1,), (1,)), ((), ())),
                                preferred_element_type=jnp.float32)
        s = s * (1.0 / math.sqrt(DH))
        m = jnp.max(s, axis=1, keepdims=True)
        p = jnp.exp(s - m).astype(jnp.bfloat16)
        o = jnp.dot(p, vh, preferred_element_type=jnp.float32)
        denom = jnp.sum(p.astype(jnp.float32), axis=1, keepdims=True)
        outs.append(o / denom)
    o_ref[...] = jnp.concatenate(outs, axis=1)


def _proj_router_body(o_ref, src_ref, wo_ref, g2_ref, b2_ref, selw_ref,
                      x_ref, x2_ref, gate_ref):
    xb = src_ref[...] + jnp.dot(o_ref[...].astype(jnp.bfloat16), wo_ref[...],
                                preferred_element_type=jnp.float32)
    x_ref[...] = xb
    nx = _ln(xb, g2_ref[...], b2_ref[...])
    x2_ref[...] = nx.astype(jnp.bfloat16)
    logits = jnp.dot(nx, selw_ref[...], preferred_element_type=jnp.float32)
    sel = jax.nn.sigmoid(logits)
    iota = jax.lax.broadcasted_iota(jnp.int32, (SB, E), 1)
    m1 = jnp.max(sel, axis=1, keepdims=True)
    i1 = jnp.min(jnp.where(sel == m1, iota, E), axis=1, keepdims=True)
    masked = jnp.where(iota == i1, -jnp.inf, sel)
    m2 = jnp.max(masked, axis=1, keepdims=True)
    i2 = jnp.min(jnp.where(masked == m2, iota, E), axis=1, keepdims=True)
    gate = jnp.where(iota == i1, m1, 0.0) + jnp.where(iota == i2, m2, 0.0)
    gate_ref[...] = gate


def _moe_body(x_ref, x2_ref, gate_ref, keys_ref, values_ref, out_ref):
    e = pl.program_id(0)

    @pl.when(e == 0)
    def _():
        out_ref[...] = x_ref[...]

    onehot = (jax.lax.broadcasted_iota(jnp.int32, (E, 1), 0) == e
              ).astype(jnp.float32)
    g = jnp.dot(gate_ref[...], onehot,
                preferred_element_type=jnp.float32)
    h = jnp.dot(x2_ref[...], keys_ref[0], preferred_element_type=jnp.float32)
    h = (jnp.maximum(h, 0.0) * g).astype(jnp.bfloat16)
    out_ref[...] += jnp.dot(h, values_ref[0],
                            preferred_element_type=jnp.float32)


def _rope_tables():
    pos = jnp.arange(S, dtype=jnp.float32)
    half = NROT // 2
    inv_freq = ROPE_BASE ** (-jnp.arange(half, dtype=jnp.float32) / half)
    ang = pos[:, None] * inv_freq[None, :]
    cos = jnp.repeat(jnp.cos(ang), 2, axis=1)
    sin = jnp.repeat(jnp.sin(ang), 2, axis=1)
    sign = jnp.where(jnp.arange(NROT) % 2 == 0, -1.0, 1.0)
    c = jnp.concatenate([cos, jnp.ones((S, DH - NROT))], axis=1)
    s = jnp.concatenate([sin * sign, jnp.zeros((S, DH - NROT))], axis=1)
    return c.astype(jnp.float32), s.astype(jnp.float32)


@jax.jit
def kernel(src, ln1_g, ln1_b, ln2_g, ln2_b, Wq, Wk, Wv, Wo, sel_w, keys,
           values):
    x0 = src.reshape(S, D)
    ctab, stab = _rope_tables()
    g1 = ln1_g.reshape(1, D)
    b1 = ln1_b.reshape(1, D)
    g2 = ln2_g.reshape(1, D)
    b2 = ln2_b.reshape(1, D)

    full = pl.BlockSpec((D, D), lambda i: (0, 0))
    row = pl.BlockSpec((1, D), lambda i: (0, 0))
    sblk = pl.BlockSpec((SB, D), lambda i: (i, 0))
    rblk = pl.BlockSpec((SB, DH), lambda i: (i, 0))

    q, k, v = pl.pallas_call(
        _qkv_body,
        grid=(NSB,),
        in_specs=[sblk, row, row, full, full, full, rblk, rblk],
        out_specs=[sblk, sblk, sblk],
        out_shape=[jax.ShapeDtypeStruct((S, D), jnp.float32)] * 3,
    )(x0, g1, b1, Wq.astype(jnp.bfloat16), Wk.astype(jnp.bfloat16),
      Wv.astype(jnp.bfloat16), ctab, stab)

    o = pl.pallas_call(
        _attn_body,
        grid=(NSB,),
        in_specs=[
            sblk,
            pl.BlockSpec((S, D), lambda i: (0, 0)),
            pl.BlockSpec((S, D), lambda i: (0, 0)),
        ],
        out_specs=sblk,
        out_shape=jax.ShapeDtypeStruct((S, D), jnp.float32),
    )(q, k, v)

    x, x2, gate = pl.pallas_call(
        _proj_router_body,
        grid=(NSB,),
        in_specs=[sblk, sblk, full, row, row,
                  pl.BlockSpec((D, E), lambda i: (0, 0))],
        out_specs=[sblk, sblk, pl.BlockSpec((SB, E), lambda i: (i, 0))],
        out_shape=[
            jax.ShapeDtypeStruct((S, D), jnp.float32),
            jax.ShapeDtypeStruct((S, D), jnp.bfloat16),
            jax.ShapeDtypeStruct((S, E), jnp.float32),
        ],
    )(o, x0, Wo.astype(jnp.bfloat16), g2, b2, sel_w)

    out = pl.pallas_call(
        _moe_body,
        grid=(E,),
        in_specs=[
            pl.BlockSpec((S, D), lambda e: (0, 0)),
            pl.BlockSpec((S, D), lambda e: (0, 0)),
            pl.BlockSpec((S, E), lambda e: (0, 0)),
            pl.BlockSpec((1, D, F), lambda e: (e, 0, 0)),
            pl.BlockSpec((1, F, D), lambda e: (e, 0, 0)),
        ],
        out_specs=pl.BlockSpec((S, D), lambda e: (0, 0)),
        out_shape=jax.ShapeDtypeStruct((S, D), jnp.float32),
    )(x, x2, gate, keys.astype(jnp.bfloat16), values.astype(jnp.bfloat16))

    return out.reshape(1, S, D)


# speedup vs baseline: 1.4490x; 1.4490x over previous
import functools
import math

import jax
import jax.numpy as jnp
from jax import lax
from jax.experimental import pallas as pl
from jax.experimental.pallas import tpu as pltpu
from jax.experimental.pallas import tpu_sc as plsc

D = 1024
H = 16
DH = D // H
NROT = DH // 2
E = 64
F = 128
K = 2
S = 2048
ROPE_BASE = 10000.0
SB = 256
NSB = S // SB
T = 128
NT = 96
P = NT * T
NW = 32
TPW = S // NW
CH = 32


def _ln(x, g, b):
    m = jnp.mean(x, axis=-1, keepdims=True)
    v = jnp.mean((x - m) ** 2, axis=-1, keepdims=True)
    return (x - m) * jax.lax.rsqrt(v + 1e-5) * g + b


def _qkv_body(x_ref, g_ref, b_ref, wq_ref, wk_ref, wv_ref, c_ref, s_ref,
              q_ref, k_ref, v_ref):
    xb = x_ref[...]
    nx = _ln(xb, g_ref[...], b_ref[...])
    q = jnp.dot(nx, wq_ref[...], preferred_element_type=jnp.float32)
    k = jnp.dot(nx, wk_ref[...], preferred_element_type=jnp.float32)
    v = jnp.dot(nx, wv_ref[...], preferred_element_type=jnp.float32)
    cb = c_ref[...]
    sb = s_ref[...]
    cfull = jnp.concatenate([cb] * H, axis=1)
    sfull = jnp.concatenate([sb] * H, axis=1)
    lane = jax.lax.broadcasted_iota(jnp.int32, (SB, D), 1)
    even = (lane % 2) == 0
    qs = jnp.where(even, jnp.roll(q, -1, axis=1), jnp.roll(q, 1, axis=1))
    ks = jnp.where(even, jnp.roll(k, -1, axis=1), jnp.roll(k, 1, axis=1))
    q_ref[...] = q * cfull + qs * sfull
    k_ref[...] = k * cfull + ks * sfull
    v_ref[...] = v


def _attn_body(q_ref, k_ref, v_ref, o_ref):
    qb = q_ref[...]
    kb = k_ref[...]
    vb = v_ref[...]
    outs = []
    for h in range(H):
        qh = qb[:, h * DH:(h + 1) * DH]
        kh = kb[:, h * DH:(h + 1) * DH]
        vh = vb[:, h * DH:(h + 1) * DH]
        s = jax.lax.dot_general(qh, kh, (((1,), (1,)), ((), ())),
                                preferred_element_type=jnp.float32)
        s = s * (1.0 / math.sqrt(DH))
        m = jnp.max(s, axis=1, keepdims=True)
        p = jnp.exp(s - m)
        denom = jnp.sum(p, axis=1, keepdims=True)
        o = jnp.dot(p, vh, preferred_element_type=jnp.float32)
        outs.append(o / denom)
    o_ref[...] = jnp.concatenate(outs, axis=1)


def _proj_router_body(o_ref, src_ref, wo_ref, g2_ref, b2_ref, selw_ref,
                      x_ref, x2_ref, gate_ref):
    xb = src_ref[...] + jnp.dot(o_ref[...], wo_ref[...],
                                preferred_element_type=jnp.float32)
    x_ref[...] = xb
    nx = _ln(xb, g2_ref[...], b2_ref[...])
    x2_ref[...] = nx
    logits = jnp.dot(nx, selw_ref[...], preferred_element_type=jnp.float32)
    sel = jax.nn.sigmoid(logits)
    iota = jax.lax.broadcasted_iota(jnp.int32, (SB, E), 1)
    m1 = jnp.max(sel, axis=1, keepdims=True)
    i1 = jnp.min(jnp.where(sel == m1, iota, E), axis=1, keepdims=True)
    masked = jnp.where(iota == i1, -jnp.inf, sel)
    m2 = jnp.max(masked, axis=1, keepdims=True)
    i2 = jnp.min(jnp.where(masked == m2, iota, E), axis=1, keepdims=True)
    gate = jnp.where(iota == i1, m1, 0.0) + jnp.where(iota == i2, m2, 0.0)
    gate_ref[...] = gate


def _sort_body(gate_ref, pos1_ref, pos2_ref, gv1_ref, gv2_ref, te_ref):
    g = gate_ref[...]
    iota_e = jax.lax.broadcasted_iota(jnp.int32, (S, E), 1)
    m1 = jnp.max(g, axis=1, keepdims=True)
    i1 = jnp.min(jnp.where(g == m1, iota_e, E), axis=1, keepdims=True)
    gm = jnp.where(iota_e == i1, -1.0, g)
    m2 = jnp.max(gm, axis=1, keepdims=True)
    i2 = jnp.min(jnp.where(gm == m2, iota_e, E), axis=1, keepdims=True)
    oh1 = (iota_e == i1).astype(jnp.int32)
    oh2 = (iota_e == i2).astype(jnp.int32)

    row = jax.lax.broadcasted_iota(jnp.int32, (S, E), 0)

    def csum_rows(x):
        c = x
        sh = 1
        while sh < S:
            c = c + jnp.where(row >= sh, jnp.roll(c, sh, axis=0), 0)
            sh *= 2
        return c

    c1 = csum_rows(oh1)
    c2 = csum_rows(oh2)
    cnt1 = c1[S - 1:S, :]
    cnt2 = c2[S - 1:S, :]
    pc = ((cnt1 + cnt2 + (T - 1)) // T) * T
    lane1 = jax.lax.broadcasted_iota(jnp.int32, (1, E), 1)
    offi = pc
    sh = 1
    while sh < E:
        offi = offi + jnp.where(lane1 >= sh, jnp.roll(offi, sh, axis=1), 0)
        sh *= 2
    off = offi - pc

    rank1 = jnp.sum((c1 - 1) * oh1, axis=1, keepdims=True)
    rank2 = jnp.sum((c2 - 1) * oh2, axis=1, keepdims=True)
    off1 = jnp.sum(oh1 * off, axis=1, keepdims=True)
    off2 = jnp.sum(oh2 * off, axis=1, keepdims=True)
    cnt1g = jnp.sum(oh2 * cnt1, axis=1, keepdims=True)
    pos1_ref[...] = off1 + rank1
    pos2_ref[...] = off2 + cnt1g + rank2
    gv1_ref[...] = m1
    gv2_ref[...] = m2

    tstart = jax.lax.broadcasted_iota(jnp.int32, (NT, 1), 0) * T
    ends = off + pc
    cmp = (ends <= tstart).astype(jnp.int32)
    te = jnp.minimum(jnp.sum(cmp, axis=1, keepdims=True), E - 1)
    te_ref[...] = te


def _moe_mm_body(te_ref, xs_ref, k_ref, v_ref, ys_ref):
    del te_ref
    h = jnp.dot(xs_ref[...], k_ref[0], preferred_element_type=jnp.float32)
    h = jnp.maximum(h, 0.0)
    ys_ref[...] = jnp.dot(h, v_ref[0], preferred_element_type=jnp.float32)


@functools.lru_cache(maxsize=None)
def _make_sc_dispatch():
    mesh = plsc.VectorSubcoreMesh(core_axis_name="c", subcore_axis_name="s")

    @functools.partial(
        pl.kernel,
        mesh=mesh,
        out_type=jax.ShapeDtypeStruct((P, D), jnp.float32),
        scratch_types=[
            pltpu.VMEM((TPW, D), jnp.float32),
            pltpu.VMEM((TPW,), jnp.int32),
            pltpu.VMEM((TPW,), jnp.int32),
            pltpu.SemaphoreType.DMA,
        ],
    )
    def dispatch(x2_hbm, pos1_hbm, pos2_hbm, xs_hbm, rows_v, idx1_v, idx2_v,
                 sem):
        wid = lax.axis_index("s") * 2 + lax.axis_index("c")
        base = wid * TPW
        pltpu.sync_copy(pos1_hbm.at[pl.ds(base, TPW)], idx1_v)
        pltpu.sync_copy(pos2_hbm.at[pl.ds(base, TPW)], idx2_v)
        pltpu.sync_copy(x2_hbm.at[pl.ds(base, TPW)], rows_v)
        pltpu.async_copy(rows_v, xs_hbm.at[idx1_v], sem).wait()
        pltpu.async_copy(rows_v, xs_hbm.at[idx2_v], sem).wait()

    return dispatch


def _sc_dispatch(x2, pos1, pos2):
    return _make_sc_dispatch()(x2, pos1, pos2)


@functools.lru_cache(maxsize=None)
def _make_sc_combine():
    mesh = plsc.VectorSubcoreMesh(core_axis_name="c", subcore_axis_name="s")

    @functools.partial(
        pl.kernel,
        mesh=mesh,
        out_type=jax.ShapeDtypeStruct((S, D), jnp.float32),
        scratch_types=[
            pltpu.VMEM((CH, D), jnp.float32),
            pltpu.VMEM((CH, D), jnp.float32),
            pltpu.VMEM((CH, D), jnp.float32),
            pltpu.VMEM((CH,), jnp.int32),
            pltpu.VMEM((CH,), jnp.int32),
            pltpu.VMEM((CH,), jnp.float32),
            pltpu.VMEM((CH,), jnp.float32),
            pltpu.SemaphoreType.DMA,
        ],
    )
    def combine(x_hbm, ys_hbm, pos1_hbm, pos2_hbm, gv1_hbm, gv2_hbm, out_hbm,
                xb_v, y1_v, y2_v, i1_v, i2_v, g1_v, g2_v, sem):
        wid = lax.axis_index("s") * 2 + lax.axis_index("c")
        for c in range(TPW // CH):
            base = (wid * (TPW // CH) + c) * CH
            pltpu.sync_copy(pos1_hbm.at[pl.ds(base, CH)], i1_v)
            pltpu.sync_copy(pos2_hbm.at[pl.ds(base, CH)], i2_v)
            pltpu.sync_copy(gv1_hbm.at[pl.ds(base, CH)], g1_v)
            pltpu.sync_copy(gv2_hbm.at[pl.ds(base, CH)], g2_v)
            pltpu.sync_copy(x_hbm.at[pl.ds(base, CH)], xb_v)
            pltpu.async_copy(ys_hbm.at[i1_v], y1_v, sem).wait()
            pltpu.async_copy(ys_hbm.at[i2_v], y2_v, sem).wait()

            for jb in range(CH // 16):
                g1vec = g1_v[pl.ds(jb * 16, 16)]
                g2vec = g2_v[pl.ds(jb * 16, 16)]
                for l in range(16):
                    j = jb * 16 + l
                    g1 = g1vec[l]
                    g2 = g2vec[l]

                    def inner(k2, carry2, j=j, g1=g1, g2=g2):
                        sl = pl.ds(k2 * 16, 16)
                        xb_v[j, sl] = (xb_v[j, sl] + g1 * y1_v[j, sl]
                                       + g2 * y2_v[j, sl])
                        return carry2

                    lax.fori_loop(0, D // 16, inner, 0)
            pltpu.sync_copy(xb_v, out_hbm.at[pl.ds(base, CH)])

    return combine


def _sc_combine(x, ys, pos1, pos2, gv1, gv2):
    return _make_sc_combine()(x, ys, pos1, pos2, gv1, gv2)


def _rope_tables():
    pos = jnp.arange(S, dtype=jnp.float32)
    half = NROT // 2
    inv_freq = ROPE_BASE ** (-jnp.arange(half, dtype=jnp.float32) / half)
    ang = pos[:, None] * inv_freq[None, :]
    cos = jnp.repeat(jnp.cos(ang), 2, axis=1)
    sin = jnp.repeat(jnp.sin(ang), 2, axis=1)
    sign = jnp.where(jnp.arange(NROT) % 2 == 0, -1.0, 1.0)
    c = jnp.concatenate([cos, jnp.ones((S, DH - NROT))], axis=1)
    s = jnp.concatenate([sin * sign, jnp.zeros((S, DH - NROT))], axis=1)
    return c.astype(jnp.float32), s.astype(jnp.float32)


@jax.jit
def kernel(src, ln1_g, ln1_b, ln2_g, ln2_b, Wq, Wk, Wv, Wo, sel_w, keys,
           values):
    x0 = src.reshape(S, D)
    ctab, stab = _rope_tables()
    g1 = ln1_g.reshape(1, D)
    b1 = ln1_b.reshape(1, D)
    g2 = ln2_g.reshape(1, D)
    b2 = ln2_b.reshape(1, D)

    full = pl.BlockSpec((D, D), lambda i: (0, 0))
    row = pl.BlockSpec((1, D), lambda i: (0, 0))
    sblk = pl.BlockSpec((SB, D), lambda i: (i, 0))
    rblk = pl.BlockSpec((SB, DH), lambda i: (i, 0))

    q, k, v = pl.pallas_call(
        _qkv_body,
        grid=(NSB,),
        in_specs=[sblk, row, row, full, full, full, rblk, rblk],
        out_specs=[sblk, sblk, sblk],
        out_shape=[jax.ShapeDtypeStruct((S, D), jnp.float32)] * 3,
    )(x0, g1, b1, Wq, Wk, Wv, ctab, stab)

    o = pl.pallas_call(
        _attn_body,
        grid=(NSB,),
        in_specs=[
            sblk,
            pl.BlockSpec((S, D), lambda i: (0, 0)),
            pl.BlockSpec((S, D), lambda i: (0, 0)),
        ],
        out_specs=sblk,
        out_shape=jax.ShapeDtypeStruct((S, D), jnp.float32),
    )(q, k, v)

    x, x2, gate = pl.pallas_call(
        _proj_router_body,
        grid=(NSB,),
        in_specs=[sblk, sblk, full, row, row,
                  pl.BlockSpec((D, E), lambda i: (0, 0))],
        out_specs=[sblk, sblk, pl.BlockSpec((SB, E), lambda i: (i, 0))],
        out_shape=[
            jax.ShapeDtypeStruct((S, D), jnp.float32),
            jax.ShapeDtypeStruct((S, D), jnp.float32),
            jax.ShapeDtypeStruct((S, E), jnp.float32),
        ],
    )(o, x0, Wo, g2, b2, sel_w)

    pos1, pos2, gv1, gv2, te = pl.pallas_call(
        _sort_body,
        grid=(1,),
        in_specs=[pl.BlockSpec((S, E), lambda i: (0, 0))],
        out_specs=[
            pl.BlockSpec((S, 1), lambda i: (0, 0)),
            pl.BlockSpec((S, 1), lambda i: (0, 0)),
            pl.BlockSpec((S, 1), lambda i: (0, 0)),
            pl.BlockSpec((S, 1), lambda i: (0, 0)),
            pl.BlockSpec((NT, 1), lambda i: (0, 0)),
        ],
        out_shape=[
            jax.ShapeDtypeStruct((S, 1), jnp.int32),
            jax.ShapeDtypeStruct((S, 1), jnp.int32),
            jax.ShapeDtypeStruct((S, 1), jnp.float32),
            jax.ShapeDtypeStruct((S, 1), jnp.float32),
            jax.ShapeDtypeStruct((NT, 1), jnp.int32),
        ],
    )(gate)

    pos1f = pos1.reshape(S)
    pos2f = pos2.reshape(S)
    gv1f = gv1.reshape(S)
    gv2f = gv2.reshape(S)
    tef = te.reshape(NT)

    xs = _sc_dispatch(x2, pos1f, pos2f)

    ys = pl.pallas_call(
        _moe_mm_body,
        grid_spec=pltpu.PrefetchScalarGridSpec(
            num_scalar_prefetch=1,
            grid=(NT,),
            in_specs=[
                pl.BlockSpec((T, D), lambda t, te_: (t, 0)),
                pl.BlockSpec((1, D, F), lambda t, te_: (te_[t], 0, 0)),
                pl.BlockSpec((1, F, D), lambda t, te_: (te_[t], 0, 0)),
            ],
            out_specs=pl.BlockSpec((T, D), lambda t, te_: (t, 0)),
        ),
        out_shape=jax.ShapeDtypeStruct((P, D), jnp.float32),
    )(tef, xs, keys, values)

    out = _sc_combine(x, ys, pos1f, pos2f, gv1f, gv2f)
    return out.reshape(1, S, D)


# speedup vs baseline: 1.5592x; 1.0761x over previous
import functools
import math

import jax
import jax.numpy as jnp
from jax import lax
from jax.experimental import pallas as pl
from jax.experimental.pallas import tpu as pltpu
from jax.experimental.pallas import tpu_sc as plsc

D = 1024
H = 16
DH = D // H
NROT = DH // 2
E = 64
F = 128
K = 2
S = 2048
ROPE_BASE = 10000.0
SB = 256
NSB = S // SB
T = 128
NT = 96
P = NT * T
NW = 32
TPW = S // NW
CH = 32


def _ln(x, g, b):
    m = jnp.mean(x, axis=-1, keepdims=True)
    v = jnp.mean((x - m) ** 2, axis=-1, keepdims=True)
    return (x - m) * jax.lax.rsqrt(v + 1e-5) * g + b


def _qkv_body(x_ref, g_ref, b_ref, wq_ref, wk_ref, wv_ref, c_ref, s_ref,
              cq_ref, sq_ref, q_ref, k_ref, v_ref):
    xb = x_ref[...]
    nx = _ln(xb, g_ref[...], b_ref[...])
    q = jnp.dot(nx, wq_ref[...], preferred_element_type=jnp.float32)
    k = jnp.dot(nx, wk_ref[...], preferred_element_type=jnp.float32)
    v = jnp.dot(nx, wv_ref[...], preferred_element_type=jnp.float32)
    cb = c_ref[...]
    sb = s_ref[...]
    cfull = jnp.concatenate([cb] * H, axis=1)
    sfull = jnp.concatenate([sb] * H, axis=1)
    cqfull = jnp.concatenate([cq_ref[...]] * H, axis=1)
    sqfull = jnp.concatenate([sq_ref[...]] * H, axis=1)
    lane = jax.lax.broadcasted_iota(jnp.int32, (SB, D), 1)
    even = (lane % 2) == 0
    qs = jnp.where(even, jnp.roll(q, -1, axis=1), jnp.roll(q, 1, axis=1))
    ks = jnp.where(even, jnp.roll(k, -1, axis=1), jnp.roll(k, 1, axis=1))
    q_ref[...] = q * cqfull + qs * sqfull
    k_ref[...] = k * cfull + ks * sfull
    v_ref[...] = v


def _attn_body(q_ref, k_ref, v_ref, o_ref):
    qb = q_ref[...]
    kb = k_ref[...]
    vb = v_ref[...]
    ones = jnp.ones((S, 1), jnp.float32)
    outs = []
    for h in range(H):
        qh = qb[:, h * DH:(h + 1) * DH]
        kh = kb[:, h * DH:(h + 1) * DH]
        vh = jnp.concatenate([vb[:, h * DH:(h + 1) * DH], ones], axis=1)
        s = jax.lax.dot_general(qh, kh, (((1,), (1,)), ((), ())),
                                preferred_element_type=jnp.float32)
        m = jnp.max(s, axis=1, keepdims=True)
        p = jnp.exp(s - m)
        o = jnp.dot(p, vh, preferred_element_type=jnp.float32)
        outs.append(o[:, :DH] * (1.0 / o[:, DH:DH + 1]))
    o_ref[...] = jnp.concatenate(outs, axis=1)


def _proj_router_body(o_ref, src_ref, wo_ref, g2_ref, b2_ref, selw_ref,
                      x_ref, x2_ref, gate_ref):
    xb = src_ref[...] + jnp.dot(o_ref[...], wo_ref[...],
                                preferred_element_type=jnp.float32)
    x_ref[...] = xb
    nx = _ln(xb, g2_ref[...], b2_ref[...])
    x2_ref[...] = nx
    logits = jnp.dot(nx, selw_ref[...], preferred_element_type=jnp.float32)
    sel = jax.nn.sigmoid(logits)
    iota = jax.lax.broadcasted_iota(jnp.int32, (SB, E), 1)
    m1 = jnp.max(sel, axis=1, keepdims=True)
    i1 = jnp.min(jnp.where(sel == m1, iota, E), axis=1, keepdims=True)
    masked = jnp.where(iota == i1, -jnp.inf, sel)
    m2 = jnp.max(masked, axis=1, keepdims=True)
    i2 = jnp.min(jnp.where(masked == m2, iota, E), axis=1, keepdims=True)
    gate = jnp.where(iota == i1, m1, 0.0) + jnp.where(iota == i2, m2, 0.0)
    gate_ref[...] = gate


def _sort_body(gate_ref, pos1_ref, pos2_ref, gv1_ref, gv2_ref, te_ref):
    g = gate_ref[...]
    iota_e = jax.lax.broadcasted_iota(jnp.int32, (S, E), 1)
    m1 = jnp.max(g, axis=1, keepdims=True)
    i1 = jnp.min(jnp.where(g == m1, iota_e, E), axis=1, keepdims=True)
    gm = jnp.where(iota_e == i1, -1.0, g)
    m2 = jnp.max(gm, axis=1, keepdims=True)
    i2 = jnp.min(jnp.where(gm == m2, iota_e, E), axis=1, keepdims=True)
    oh1 = (iota_e == i1).astype(jnp.int32)
    oh2 = (iota_e == i2).astype(jnp.int32)

    row = jax.lax.broadcasted_iota(jnp.int32, (S, E), 0)

    def csum_rows(x):
        c = x
        sh = 1
        while sh < S:
            c = c + jnp.where(row >= sh, jnp.roll(c, sh, axis=0), 0)
            sh *= 2
        return c

    c1 = csum_rows(oh1)
    c2 = csum_rows(oh2)
    cnt1 = c1[S - 1:S, :]
    cnt2 = c2[S - 1:S, :]
    pc = ((cnt1 + cnt2 + (T - 1)) // T) * T
    lane1 = jax.lax.broadcasted_iota(jnp.int32, (1, E), 1)
    offi = pc
    sh = 1
    while sh < E:
        offi = offi + jnp.where(lane1 >= sh, jnp.roll(offi, sh, axis=1), 0)
        sh *= 2
    off = offi - pc

    rank1 = jnp.sum((c1 - 1) * oh1, axis=1, keepdims=True)
    rank2 = jnp.sum((c2 - 1) * oh2, axis=1, keepdims=True)
    off1 = jnp.sum(oh1 * off, axis=1, keepdims=True)
    off2 = jnp.sum(oh2 * off, axis=1, keepdims=True)
    cnt1g = jnp.sum(oh2 * cnt1, axis=1, keepdims=True)
    pos1_ref[...] = off1 + rank1
    pos2_ref[...] = off2 + cnt1g + rank2
    gv1_ref[...] = m1
    gv2_ref[...] = m2

    tstart = jax.lax.broadcasted_iota(jnp.int32, (NT, 1), 0) * T
    ends = off + pc
    cmp = (ends <= tstart).astype(jnp.int32)
    te = jnp.minimum(jnp.sum(cmp, axis=1, keepdims=True), E - 1)
    te_ref[...] = te


def _moe_mm_body(te_ref, xs_ref, k_ref, v_ref, ys_ref):
    del te_ref
    h = jnp.dot(xs_ref[...], k_ref[0], preferred_element_type=jnp.float32)
    h = jnp.maximum(h, 0.0)
    ys_ref[...] = jnp.dot(h, v_ref[0], preferred_element_type=jnp.float32)


@functools.lru_cache(maxsize=None)
def _make_sc_dispatch():
    mesh = plsc.VectorSubcoreMesh(core_axis_name="c", subcore_axis_name="s")

    @functools.partial(
        pl.kernel,
        mesh=mesh,
        out_type=jax.ShapeDtypeStruct((P, D), jnp.float32),
        scratch_types=[
            pltpu.VMEM((TPW, D), jnp.float32),
            pltpu.VMEM((TPW,), jnp.int32),
            pltpu.VMEM((TPW,), jnp.int32),
            pltpu.SemaphoreType.DMA,
        ],
    )
    def dispatch(x2_hbm, pos1_hbm, pos2_hbm, xs_hbm, rows_v, idx1_v, idx2_v,
                 sem):
        wid = lax.axis_index("s") * 2 + lax.axis_index("c")
        base = wid * TPW
        pltpu.sync_copy(pos1_hbm.at[pl.ds(base, TPW)], idx1_v)
        pltpu.sync_copy(pos2_hbm.at[pl.ds(base, TPW)], idx2_v)
        pltpu.sync_copy(x2_hbm.at[pl.ds(base, TPW)], rows_v)
        pltpu.async_copy(rows_v, xs_hbm.at[idx1_v], sem).wait()
        pltpu.async_copy(rows_v, xs_hbm.at[idx2_v], sem).wait()

    return dispatch


def _sc_dispatch(x2, pos1, pos2):
    return _make_sc_dispatch()(x2, pos1, pos2)


@functools.lru_cache(maxsize=None)
def _make_sc_combine():
    mesh = plsc.VectorSubcoreMesh(core_axis_name="c", subcore_axis_name="s")

    @functools.partial(
        pl.kernel,
        mesh=mesh,
        out_type=jax.ShapeDtypeStruct((S, D), jnp.float32),
        scratch_types=[
            pltpu.VMEM((CH, D), jnp.float32),
            pltpu.VMEM((CH, D), jnp.float32),
            pltpu.VMEM((CH, D), jnp.float32),
            pltpu.VMEM((CH,), jnp.int32),
            pltpu.VMEM((CH,), jnp.int32),
            pltpu.VMEM((CH,), jnp.float32),
            pltpu.VMEM((CH,), jnp.float32),
            pltpu.SemaphoreType.DMA,
        ],
    )
    def combine(x_hbm, ys_hbm, pos1_hbm, pos2_hbm, gv1_hbm, gv2_hbm, out_hbm,
                xb_v, y1_v, y2_v, i1_v, i2_v, g1_v, g2_v, sem):
        wid = lax.axis_index("s") * 2 + lax.axis_index("c")
        for c in range(TPW // CH):
            base = (wid * (TPW // CH) + c) * CH
            pltpu.sync_copy(pos1_hbm.at[pl.ds(base, CH)], i1_v)
            pltpu.sync_copy(pos2_hbm.at[pl.ds(base, CH)], i2_v)
            pltpu.sync_copy(gv1_hbm.at[pl.ds(base, CH)], g1_v)
            pltpu.sync_copy(gv2_hbm.at[pl.ds(base, CH)], g2_v)
            cp1 = pltpu.async_copy(ys_hbm.at[i1_v], y1_v, sem)
            cp2 = pltpu.async_copy(ys_hbm.at[i2_v], y2_v, sem)
            pltpu.sync_copy(x_hbm.at[pl.ds(base, CH)], xb_v)
            cp1.wait()
            cp2.wait()

            for jb in range(CH // 16):
                g1vec = g1_v[pl.ds(jb * 16, 16)]
                g2vec = g2_v[pl.ds(jb * 16, 16)]
                for l in range(16):
                    j = jb * 16 + l
                    g1 = g1vec[l]
                    g2 = g2vec[l]

                    def inner(k2, carry2, j=j, g1=g1, g2=g2):
                        for u in range(4):
                            sl = pl.ds(k2 * 64 + u * 16, 16)
                            xb_v[j, sl] = (xb_v[j, sl] + g1 * y1_v[j, sl]
                                           + g2 * y2_v[j, sl])
                        return carry2

                    lax.fori_loop(0, D // 64, inner, 0)
            pltpu.sync_copy(xb_v, out_hbm.at[pl.ds(base, CH)])

    return combine


def _sc_combine(x, ys, pos1, pos2, gv1, gv2):
    return _make_sc_combine()(x, ys, pos1, pos2, gv1, gv2)


def _rope_tables():
    pos = jnp.arange(S, dtype=jnp.float32)
    half = NROT // 2
    inv_freq = ROPE_BASE ** (-jnp.arange(half, dtype=jnp.float32) / half)
    ang = pos[:, None] * inv_freq[None, :]
    cos = jnp.repeat(jnp.cos(ang), 2, axis=1)
    sin = jnp.repeat(jnp.sin(ang), 2, axis=1)
    sign = jnp.where(jnp.arange(NROT) % 2 == 0, -1.0, 1.0)
    c = jnp.concatenate([cos, jnp.ones((S, DH - NROT))], axis=1)
    s = jnp.concatenate([sin * sign, jnp.zeros((S, DH - NROT))], axis=1)
    return c.astype(jnp.float32), s.astype(jnp.float32)


@jax.jit
def kernel(src, ln1_g, ln1_b, ln2_g, ln2_b, Wq, Wk, Wv, Wo, sel_w, keys,
           values):
    x0 = src.reshape(S, D)
    ctab, stab = _rope_tables()
    g1 = ln1_g.reshape(1, D)
    b1 = ln1_b.reshape(1, D)
    g2 = ln2_g.reshape(1, D)
    b2 = ln2_b.reshape(1, D)

    full = pl.BlockSpec((D, D), lambda i: (0, 0))
    row = pl.BlockSpec((1, D), lambda i: (0, 0))
    sblk = pl.BlockSpec((SB, D), lambda i: (i, 0))
    rblk = pl.BlockSpec((SB, DH), lambda i: (i, 0))

    scale = 1.0 / math.sqrt(DH)
    q, k, v = pl.pallas_call(
        _qkv_body,
        grid=(NSB,),
        in_specs=[sblk, row, row, full, full, full, rblk, rblk, rblk, rblk],
        out_specs=[sblk, sblk, sblk],
        out_shape=[jax.ShapeDtypeStruct((S, D), jnp.float32)] * 3,
    )(x0, g1, b1, Wq, Wk, Wv, ctab, stab, ctab * scale, stab * scale)

    o = pl.pallas_call(
        _attn_body,
        grid=(NSB,),
        in_specs=[
            sblk,
            pl.BlockSpec((S, D), lambda i: (0, 0)),
            pl.BlockSpec((S, D), lambda i: (0, 0)),
        ],
        out_specs=sblk,
        out_shape=jax.ShapeDtypeStruct((S, D), jnp.float32),
    )(q, k, v)

    x, x2, gate = pl.pallas_call(
        _proj_router_body,
        grid=(NSB,),
        in_specs=[sblk, sblk, full, row, row,
                  pl.BlockSpec((D, E), lambda i: (0, 0))],
        out_specs=[sblk, sblk, pl.BlockSpec((SB, E), lambda i: (i, 0))],
        out_shape=[
            jax.ShapeDtypeStruct((S, D), jnp.float32),
            jax.ShapeDtypeStruct((S, D), jnp.float32),
            jax.ShapeDtypeStruct((S, E), jnp.float32),
        ],
    )(o, x0, Wo, g2, b2, sel_w)

    pos1, pos2, gv1, gv2, te = pl.pallas_call(
        _sort_body,
        grid=(1,),
        in_specs=[pl.BlockSpec((S, E), lambda i: (0, 0))],
        out_specs=[
            pl.BlockSpec((S, 1), lambda i: (0, 0)),
            pl.BlockSpec((S, 1), lambda i: (0, 0)),
            pl.BlockSpec((S, 1), lambda i: (0, 0)),
            pl.BlockSpec((S, 1), lambda i: (0, 0)),
            pl.BlockSpec((NT, 1), lambda i: (0, 0)),
        ],
        out_shape=[
            jax.ShapeDtypeStruct((S, 1), jnp.int32),
            jax.ShapeDtypeStruct((S, 1), jnp.int32),
            jax.ShapeDtypeStruct((S, 1), jnp.float32),
            jax.ShapeDtypeStruct((S, 1), jnp.float32),
            jax.ShapeDtypeStruct((NT, 1), jnp.int32),
        ],
    )(gate)

    pos1f = pos1.reshape(S)
    pos2f = pos2.reshape(S)
    gv1f = gv1.reshape(S)
    gv2f = gv2.reshape(S)
    tef = te.reshape(NT)

    xs = _sc_dispatch(x2, pos1f, pos2f)

    ys = pl.pallas_call(
        _moe_mm_body,
        grid_spec=pltpu.PrefetchScalarGridSpec(
            num_scalar_prefetch=1,
            grid=(NT,),
            in_specs=[
                pl.BlockSpec((T, D), lambda t, te_: (t, 0)),
                pl.BlockSpec((1, D, F), lambda t, te_: (te_[t], 0, 0)),
                pl.BlockSpec((1, F, D), lambda t, te_: (te_[t], 0, 0)),
            ],
            out_specs=pl.BlockSpec((T, D), lambda t, te_: (t, 0)),
        ),
        out_shape=jax.ShapeDtypeStruct((P, D), jnp.float32),
    )(tef, xs, keys, values)

    out = _sc_combine(x, ys, pos1f, pos2f, gv1f, gv2f)
    return out.reshape(1, S, D)


# speedup vs baseline: 1.5921x; 1.0211x over previous
import functools
import math

import jax
import jax.numpy as jnp
from jax import lax
from jax.experimental import pallas as pl
from jax.experimental.pallas import tpu as pltpu
from jax.experimental.pallas import tpu_sc as plsc

D = 1024
H = 16
DH = D // H
NROT = DH // 2
E = 64
F = 128
K = 2
S = 2048
ROPE_BASE = 10000.0
SB = 256
NSB = S // SB
T = 128
NT = 96
P = NT * T
NW = 32
TPW = S // NW
CH = 32


def _ln(x, g, b):
    m = jnp.mean(x, axis=-1, keepdims=True)
    v = jnp.mean((x - m) ** 2, axis=-1, keepdims=True)
    return (x - m) * jax.lax.rsqrt(v + 1e-5) * g + b


def _qkv_body(x_ref, g_ref, b_ref, wq_ref, wk_ref, wv_ref, c_ref, s_ref,
              cq_ref, sq_ref, q_ref, k_ref, v_ref):
    xb = x_ref[...]
    nx = _ln(xb, g_ref[...], b_ref[...])
    q = jnp.dot(nx, wq_ref[...], preferred_element_type=jnp.float32)
    k = jnp.dot(nx, wk_ref[...], preferred_element_type=jnp.float32)
    v = jnp.dot(nx, wv_ref[...], preferred_element_type=jnp.float32)
    cb = c_ref[...]
    sb = s_ref[...]
    cfull = jnp.concatenate([cb] * H, axis=1)
    sfull = jnp.concatenate([sb] * H, axis=1)
    cqfull = jnp.concatenate([cq_ref[...]] * H, axis=1)
    sqfull = jnp.concatenate([sq_ref[...]] * H, axis=1)
    lane = jax.lax.broadcasted_iota(jnp.int32, (SB, D), 1)
    even = (lane % 2) == 0
    qs = jnp.where(even, jnp.roll(q, -1, axis=1), jnp.roll(q, 1, axis=1))
    ks = jnp.where(even, jnp.roll(k, -1, axis=1), jnp.roll(k, 1, axis=1))
    q_ref[...] = q * cqfull + qs * sqfull
    k_ref[...] = k * cfull + ks * sfull
    v_ref[...] = v


def _attn_body(q_ref, k_ref, v_ref, o_ref):
    qb = q_ref[...]
    kb = k_ref[...]
    vb = v_ref[...]
    ones = jnp.ones((S, 1), jnp.float32)
    outs = []
    for h in range(H):
        qh = qb[:, h * DH:(h + 1) * DH]
        kh = kb[:, h * DH:(h + 1) * DH]
        vh = jnp.concatenate([vb[:, h * DH:(h + 1) * DH], ones], axis=1)
        s = jax.lax.dot_general(qh, kh, (((1,), (1,)), ((), ())),
                                preferred_element_type=jnp.float32)
        m = jnp.max(s, axis=1, keepdims=True)
        p = jnp.exp(s - m)
        o = jnp.dot(p, vh, preferred_element_type=jnp.float32)
        outs.append(o[:, :DH] * (1.0 / o[:, DH:DH + 1]))
    o_ref[...] = jnp.concatenate(outs, axis=1)


def _proj_router_sort_body(o_ref, src_ref, wo_ref, g2_ref, b2_ref, selw_ref,
                           x_ref, x2_ref, pos1_ref, pos2_ref, gv1_ref,
                           gv2_ref, te_ref):
    xb = src_ref[...] + jnp.dot(o_ref[...], wo_ref[...],
                                preferred_element_type=jnp.float32)
    x_ref[...] = xb
    nx = _ln(xb, g2_ref[...], b2_ref[...])
    x2_ref[...] = nx
    logits = jnp.dot(nx, selw_ref[...], preferred_element_type=jnp.float32)
    sel = jax.nn.sigmoid(logits)
    iota_e = jax.lax.broadcasted_iota(jnp.int32, (S, E), 1)
    m1 = jnp.max(sel, axis=1, keepdims=True)
    i1 = jnp.min(jnp.where(sel == m1, iota_e, E), axis=1, keepdims=True)
    gm = jnp.where(iota_e == i1, -jnp.inf, sel)
    m2 = jnp.max(gm, axis=1, keepdims=True)
    i2 = jnp.min(jnp.where(gm == m2, iota_e, E), axis=1, keepdims=True)
    oh1 = (iota_e == i1).astype(jnp.int32)
    oh2 = (iota_e == i2).astype(jnp.int32)

    row = jax.lax.broadcasted_iota(jnp.int32, (S, E), 0)

    def csum_rows(x):
        c = x
        sh = 1
        while sh < S:
            c = c + jnp.where(row >= sh, jnp.roll(c, sh, axis=0), 0)
            sh *= 2
        return c

    c1 = csum_rows(oh1)
    c2 = csum_rows(oh2)
    cnt1 = c1[S - 1:S, :]
    cnt2 = c2[S - 1:S, :]
    pc = ((cnt1 + cnt2 + (T - 1)) // T) * T
    lane1 = jax.lax.broadcasted_iota(jnp.int32, (1, E), 1)
    offi = pc
    sh = 1
    while sh < E:
        offi = offi + jnp.where(lane1 >= sh, jnp.roll(offi, sh, axis=1), 0)
        sh *= 2
    off = offi - pc

    rank1 = jnp.sum((c1 - 1) * oh1, axis=1, keepdims=True)
    rank2 = jnp.sum((c2 - 1) * oh2, axis=1, keepdims=True)
    off1 = jnp.sum(oh1 * off, axis=1, keepdims=True)
    off2 = jnp.sum(oh2 * off, axis=1, keepdims=True)
    cnt1g = jnp.sum(oh2 * cnt1, axis=1, keepdims=True)
    pos1_ref[...] = off1 + rank1
    pos2_ref[...] = off2 + cnt1g + rank2
    gv1_ref[...] = m1
    gv2_ref[...] = m2

    tstart = jax.lax.broadcasted_iota(jnp.int32, (NT, 1), 0) * T
    ends = off + pc
    cmp = (ends <= tstart).astype(jnp.int32)
    te = jnp.minimum(jnp.sum(cmp, axis=1, keepdims=True), E - 1)
    te_ref[...] = te


def _moe_mm_body(te_ref, xs_ref, k_ref, v_ref, ys_ref):
    del te_ref
    h = jnp.dot(xs_ref[...], k_ref[0], preferred_element_type=jnp.float32)
    h = jnp.maximum(h, 0.0)
    ys_ref[...] = jnp.dot(h, v_ref[0], preferred_element_type=jnp.float32)


@functools.lru_cache(maxsize=None)
def _make_sc_dispatch():
    mesh = plsc.VectorSubcoreMesh(core_axis_name="c", subcore_axis_name="s")

    @functools.partial(
        pl.kernel,
        mesh=mesh,
        out_type=jax.ShapeDtypeStruct((P, D), jnp.float32),
        scratch_types=[
            pltpu.VMEM((TPW, D), jnp.float32),
            pltpu.VMEM((TPW,), jnp.int32),
            pltpu.VMEM((TPW,), jnp.int32),
            pltpu.SemaphoreType.DMA,
        ],
    )
    def dispatch(x2_hbm, pos1_hbm, pos2_hbm, xs_hbm, rows_v, idx1_v, idx2_v,
                 sem):
        wid = lax.axis_index("s") * 2 + lax.axis_index("c")
        base = wid * TPW
        pltpu.sync_copy(pos1_hbm.at[pl.ds(base, TPW)], idx1_v)
        pltpu.sync_copy(pos2_hbm.at[pl.ds(base, TPW)], idx2_v)
        pltpu.sync_copy(x2_hbm.at[pl.ds(base, TPW)], rows_v)
        pltpu.async_copy(rows_v, xs_hbm.at[idx1_v], sem).wait()
        pltpu.async_copy(rows_v, xs_hbm.at[idx2_v], sem).wait()

    return dispatch


def _sc_dispatch(x2, pos1, pos2):
    return _make_sc_dispatch()(x2, pos1, pos2)


@functools.lru_cache(maxsize=None)
def _make_sc_combine():
    mesh = plsc.VectorSubcoreMesh(core_axis_name="c", subcore_axis_name="s")

    @functools.partial(
        pl.kernel,
        mesh=mesh,
        out_type=jax.ShapeDtypeStruct((S, D), jnp.float32),
        scratch_types=[
            pltpu.VMEM((CH, D), jnp.float32),
            pltpu.VMEM((CH, D), jnp.float32),
            pltpu.VMEM((CH, D), jnp.float32),
            pltpu.VMEM((CH,), jnp.int32),
            pltpu.VMEM((CH,), jnp.int32),
            pltpu.VMEM((CH,), jnp.float32),
            pltpu.VMEM((CH,), jnp.float32),
            pltpu.SemaphoreType.DMA,
        ],
    )
    def combine(x_hbm, ys_hbm, pos1_hbm, pos2_hbm, gv1_hbm, gv2_hbm, out_hbm,
                xb_v, y1_v, y2_v, i1_v, i2_v, g1_v, g2_v, sem):
        wid = lax.axis_index("s") * 2 + lax.axis_index("c")
        for c in range(TPW // CH):
            base = (wid * (TPW // CH) + c) * CH
            pltpu.sync_copy(pos1_hbm.at[pl.ds(base, CH)], i1_v)
            pltpu.sync_copy(pos2_hbm.at[pl.ds(base, CH)], i2_v)
            pltpu.sync_copy(gv1_hbm.at[pl.ds(base, CH)], g1_v)
            pltpu.sync_copy(gv2_hbm.at[pl.ds(base, CH)], g2_v)
            cp1 = pltpu.async_copy(ys_hbm.at[i1_v], y1_v, sem)
            cp2 = pltpu.async_copy(ys_hbm.at[i2_v], y2_v, sem)
            pltpu.sync_copy(x_hbm.at[pl.ds(base, CH)], xb_v)
            cp1.wait()
            cp2.wait()

            for jb in range(CH // 16):
                g1vec = g1_v[pl.ds(jb * 16, 16)]
                g2vec = g2_v[pl.ds(jb * 16, 16)]
                for l in range(16):
                    j = jb * 16 + l
                    g1 = g1vec[l]
                    g2 = g2vec[l]

                    def inner(k2, carry2, j=j, g1=g1, g2=g2):
                        for u in range(4):
                            sl = pl.ds(k2 * 64 + u * 16, 16)
                            xb_v[j, sl] = (xb_v[j, sl] + g1 * y1_v[j, sl]
                                           + g2 * y2_v[j, sl])
                        return carry2

                    lax.fori_loop(0, D // 64, inner, 0)
            pltpu.sync_copy(xb_v, out_hbm.at[pl.ds(base, CH)])

    return combine


def _sc_combine(x, ys, pos1, pos2, gv1, gv2):
    return _make_sc_combine()(x, ys, pos1, pos2, gv1, gv2)


def _rope_tables():
    pos = jnp.arange(S, dtype=jnp.float32)
    half = NROT // 2
    inv_freq = ROPE_BASE ** (-jnp.arange(half, dtype=jnp.float32) / half)
    ang = pos[:, None] * inv_freq[None, :]
    cos = jnp.repeat(jnp.cos(ang), 2, axis=1)
    sin = jnp.repeat(jnp.sin(ang), 2, axis=1)
    sign = jnp.where(jnp.arange(NROT) % 2 == 0, -1.0, 1.0)
    c = jnp.concatenate([cos, jnp.ones((S, DH - NROT))], axis=1)
    s = jnp.concatenate([sin * sign, jnp.zeros((S, DH - NROT))], axis=1)
    return c.astype(jnp.float32), s.astype(jnp.float32)


@jax.jit
def kernel(src, ln1_g, ln1_b, ln2_g, ln2_b, Wq, Wk, Wv, Wo, sel_w, keys,
           values):
    x0 = src.reshape(S, D)
    ctab, stab = _rope_tables()
    g1 = ln1_g.reshape(1, D)
    b1 = ln1_b.reshape(1, D)
    g2 = ln2_g.reshape(1, D)
    b2 = ln2_b.reshape(1, D)

    full = pl.BlockSpec((D, D), lambda i: (0, 0))
    row = pl.BlockSpec((1, D), lambda i: (0, 0))
    sblk = pl.BlockSpec((SB, D), lambda i: (i, 0))
    rblk = pl.BlockSpec((SB, DH), lambda i: (i, 0))

    scale = 1.0 / math.sqrt(DH)
    q, k, v = pl.pallas_call(
        _qkv_body,
        grid=(NSB,),
        in_specs=[sblk, row, row, full, full, full, rblk, rblk, rblk, rblk],
        out_specs=[sblk, sblk, sblk],
        out_shape=[jax.ShapeDtypeStruct((S, D), jnp.float32)] * 3,
    )(x0, g1, b1, Wq, Wk, Wv, ctab, stab, ctab * scale, stab * scale)

    o = pl.pallas_call(
        _attn_body,
        grid=(NSB,),
        in_specs=[
            sblk,
            pl.BlockSpec((S, D), lambda i: (0, 0)),
            pl.BlockSpec((S, D), lambda i: (0, 0)),
        ],
        out_specs=sblk,
        out_shape=jax.ShapeDtypeStruct((S, D), jnp.float32),
    )(q, k, v)

    fullS = pl.BlockSpec((S, D), lambda i: (0, 0))
    col = pl.BlockSpec((S, 1), lambda i: (0, 0))
    x, x2, pos1, pos2, gv1, gv2, te = pl.pallas_call(
        _proj_router_sort_body,
        grid=(1,),
        in_specs=[fullS, fullS, full, row, row,
                  pl.BlockSpec((D, E), lambda i: (0, 0))],
        out_specs=[fullS, fullS, col, col, col, col,
                   pl.BlockSpec((NT, 1), lambda i: (0, 0))],
        out_shape=[
            jax.ShapeDtypeStruct((S, D), jnp.float32),
            jax.ShapeDtypeStruct((S, D), jnp.float32),
            jax.ShapeDtypeStruct((S, 1), jnp.int32),
            jax.ShapeDtypeStruct((S, 1), jnp.int32),
            jax.ShapeDtypeStruct((S, 1), jnp.float32),
            jax.ShapeDtypeStruct((S, 1), jnp.float32),
            jax.ShapeDtypeStruct((NT, 1), jnp.int32),
        ],
    )(o, x0, Wo, g2, b2, sel_w)

    pos1f = pos1.reshape(S)
    pos2f = pos2.reshape(S)
    gv1f = gv1.reshape(S)
    gv2f = gv2.reshape(S)
    tef = te.reshape(NT)

    xs = _sc_dispatch(x2, pos1f, pos2f)

    ys = pl.pallas_call(
        _moe_mm_body,
        grid_spec=pltpu.PrefetchScalarGridSpec(
            num_scalar_prefetch=1,
            grid=(NT,),
            in_specs=[
                pl.BlockSpec((T, D), lambda t, te_: (t, 0)),
                pl.BlockSpec((1, D, F), lambda t, te_: (te_[t], 0, 0)),
                pl.BlockSpec((1, F, D), lambda t, te_: (te_[t], 0, 0)),
            ],
            out_specs=pl.BlockSpec((T, D), lambda t, te_: (t, 0)),
        ),
        out_shape=jax.ShapeDtypeStruct((P, D), jnp.float32),
    )(tef, xs, keys, values)

    out = _sc_combine(x, ys, pos1f, pos2f, gv1f, gv2f)
    return out.reshape(1, S, D)
